# baseline (speedup 1.0000x reference)
import functools
import jax
import jax.numpy as jnp
from jax import lax
from jax.experimental import pallas as pl
from jax.experimental.pallas import tpu as pltpu
from jax.experimental.pallas import tpu_sc as plsc

D = 64
NC, NS = 2, 16
NW = NC * NS
CHUNK = 128
GRP = 2
NBUF = 2 * GRP


@functools.lru_cache(maxsize=None)
def _build(S, R):
    B = S * R
    assert B % (NW * CHUNK) == 0 and R % CHUNK == 0
    b_per_w = B // NW
    n_chunks = b_per_w // CHUNK
    n_outer = n_chunks // GRP
    assert n_chunks % (2 * GRP) == 0
    RT = R // CHUNK

    mesh = plsc.VectorSubcoreMesh(core_axis_name="c", subcore_axis_name="s")

    scratch = [pltpu.VMEM((n_chunks, CHUNK), jnp.int32)]
    scratch += [pltpu.VMEM((CHUNK, D), jnp.float32) for _ in range(NBUF)]
    scratch += [pltpu.VMEM((D // 8, 8, CHUNK), jnp.float32)
                for _ in range(NBUF)]
    scratch += [pltpu.SemaphoreType.DMA, pltpu.SemaphoreType.DMA]

    @functools.partial(
        pl.kernel,
        mesh=mesh,
        compiler_params=pltpu.CompilerParams(
            use_tc_tiling_on_sc=False, needs_layout_passes=False
        ),
        out_type=jax.ShapeDtypeStruct((S, D // 8, RT, 8, CHUNK), jnp.float32),
        scratch_types=scratch,
    )
    def k(idx_hbm, table_hbm, out_hbm, idx_v, *bufs):
        rows_bufs = bufs[:NBUF]
        trans_bufs = bufs[NBUF:2 * NBUF]
        gsem, wsem = bufs[2 * NBUF], bufs[2 * NBUF + 1]

        wid = lax.axis_index("s") * NC + lax.axis_index("c")
        cbase = wid * n_chunks
        pltpu.sync_copy(idx_hbm.at[wid], idx_v)

        iota16 = lax.iota(jnp.int32, 16)
        row_vecs = [iota16 + rg * 16 for rg in range(CHUNK // 16)]

        def transpose_chunk(buf):
            rows = rows_bufs[buf]
            trans = trans_bufs[buf]
            for d in range(D):
                col = iota16 * 0 + d
                for rg in range(CHUNK // 16):
                    vec = plsc.load_gather(rows, [row_vecs[rg], col])
                    trans[d // 8, d % 8, pl.ds(rg * 16, 16)] = vec

        def outer(p, carry):
            for grp in range(2):
                o = p * 2 + grp
                @pl.when(p >= 1)
                def _():
                    for j in range(GRP):
                        pltpu.make_async_copy(
                            trans_bufs[j], out_hbm.at[0, :, 0], wsem
                        ).wait()

                copies = []
                for j in range(GRP):
                    g = o * GRP + j
                    buf = grp * GRP + j
                    copies.append(
                        pltpu.async_copy(
                            table_hbm.at[idx_v.at[g]], rows_bufs[buf], gsem
                        )
                    )
                for c in copies:
                    c.wait()
                for j in range(GRP):
                    g = o * GRP + j
                    buf = grp * GRP + j
                    transpose_chunk(buf)
                    c = cbase + g
                    s = c // RT
                    tr = lax.rem(c, RT)
                    pltpu.async_copy(
                        trans_bufs[buf], out_hbm.at[s, :, tr], wsem
                    )
            return carry

        lax.fori_loop(0, n_outer // 2, outer, 0, unroll=False)
        for j in range(NBUF):
            pltpu.make_async_copy(
                trans_bufs[j], out_hbm.at[0, :, 0], wsem
            ).wait()

    return k


def kernel(x, table):
    R, S = x.shape
    xt = x.T.astype(jnp.int32)
    idx3 = xt.reshape(NW, (S * R) // (NW * CHUNK), CHUNK)
    out5 = _build(S, R)(idx3, table)
    return out5.transpose(2, 4, 0, 1, 3).reshape(R, S, D)


# speedup vs baseline: 1.6996x; 1.6996x over previous
import functools
import jax
import jax.numpy as jnp
from jax import lax
from jax.experimental import pallas as pl
from jax.experimental.pallas import tpu as pltpu
from jax.experimental.pallas import tpu_sc as plsc

D = 64
NC, NS = 2, 16
NW = NC * NS
CHUNK = 128
GRP = 2
NBUF = 2 * GRP


@functools.lru_cache(maxsize=None)
def _build(S, R):
    B = S * R
    assert B % (NW * CHUNK) == 0 and R % CHUNK == 0
    b_per_w = B // NW
    n_chunks = b_per_w // CHUNK
    n_grp = n_chunks // GRP
    assert n_chunks % (2 * GRP) == 0
    RT = R // CHUNK

    mesh = plsc.VectorSubcoreMesh(core_axis_name="c", subcore_axis_name="s")

    scratch = [pltpu.VMEM((n_chunks, CHUNK), jnp.int32)]
    scratch += [pltpu.VMEM((CHUNK, D), jnp.float32) for _ in range(NBUF)]
    scratch += [pltpu.VMEM((D // 8, 8, CHUNK), jnp.float32)
                for _ in range(NBUF)]
    scratch += [pltpu.SemaphoreType.DMA, pltpu.SemaphoreType.DMA]

    @functools.partial(
        pl.kernel,
        mesh=mesh,
        compiler_params=pltpu.CompilerParams(
            use_tc_tiling_on_sc=False, needs_layout_passes=False
        ),
        out_type=jax.ShapeDtypeStruct((S, D // 8, RT, 8, CHUNK), jnp.float32),
        scratch_types=scratch,
    )
    def k(idx_hbm, table_hbm, out_hbm, idx_v, *bufs):
        rows_bufs = bufs[:NBUF]
        trans_bufs = bufs[NBUF:2 * NBUF]
        gsem, wsem = bufs[2 * NBUF], bufs[2 * NBUF + 1]

        wid = lax.axis_index("s") * NC + lax.axis_index("c")
        cbase = wid * n_chunks
        pltpu.sync_copy(idx_hbm.at[wid], idx_v)

        iota16 = lax.iota(jnp.int32, 16)
        td_vecs, di_vecs = [], []
        for dg in range(D // 16):
            dvec = iota16 + dg * 16
            td_vecs.append(lax.shift_right_logical(dvec, 3))
            di_vecs.append(lax.bitwise_and(dvec, 7))

        zero16 = iota16 * 0

        def transpose_chunk(buf):
            rows = rows_bufs[buf]
            trans = trans_bufs[buf]

            @plsc.parallel_loop(0, CHUNK, unroll=8)
            def _(r):
                rsplat = zero16 + r
                for dg in range(D // 16):
                    vec = rows[r, pl.ds(dg * 16, 16)]
                    plsc.store_scatter(
                        trans, [td_vecs[dg], di_vecs[dg], rsplat], vec
                    )

        def gather_group(q, bufset):
            for j in range(GRP):
                pltpu.async_copy(
                    table_hbm.at[idx_v.at[q * GRP + j]],
                    rows_bufs[bufset * GRP + j],
                    gsem,
                )

        def body(q, bufset):
            @pl.when(q >= 2)
            def _():
                for j in range(GRP):
                    pltpu.make_async_copy(
                        trans_bufs[j], out_hbm.at[0, :, 0], wsem
                    ).wait()
            for j in range(GRP):
                pltpu.make_async_copy(
                    table_hbm.at[pl.ds(0, CHUNK)], rows_bufs[0], gsem
                ).wait()
            @pl.when(q + 1 < n_grp)
            def _():
                gather_group(q + 1, 1 - bufset)
            for j in range(GRP):
                buf = bufset * GRP + j
                transpose_chunk(buf)
                c = cbase + q * GRP + j
                s = c // RT
                tr = lax.rem(c, RT)
                pltpu.async_copy(
                    trans_bufs[buf], out_hbm.at[s, :, tr], wsem
                )

        def outer(p, carry):
            for grp in range(2):
                body(p * 2 + grp, grp)
            return carry

        gather_group(0, 0)
        lax.fori_loop(0, n_grp // 2, outer, 0, unroll=False)
        for j in range(2 * GRP):
            pltpu.make_async_copy(
                trans_bufs[j % GRP], out_hbm.at[0, :, 0], wsem
            ).wait()

    return k


def kernel(x, table):
    R, S = x.shape
    xt = x.T.astype(jnp.int32)
    idx3 = xt.reshape(NW, (S * R) // (NW * CHUNK), CHUNK)
    out5 = _build(S, R)(idx3, table)
    return out5.transpose(2, 4, 0, 1, 3).reshape(R, S, D)


# speedup vs baseline: 2.8688x; 1.6879x over previous
import functools
import jax
import jax.numpy as jnp
from jax import lax
from jax.experimental import pallas as pl
from jax.experimental.pallas import tpu as pltpu
from jax.experimental.pallas import tpu_sc as plsc

D = 64
NC, NS = 2, 16
NW = NC * NS
CHUNK = 128
GRP = 2
NBUF = 2 * GRP


@functools.lru_cache(maxsize=None)
def _build(S, R):
    B = S * R
    assert B % (NW * CHUNK) == 0 and R % CHUNK == 0
    b_per_w = B // NW
    n_chunks = b_per_w // CHUNK
    n_grp = n_chunks // GRP
    assert n_chunks % (2 * GRP) == 0
    RT = R // CHUNK

    mesh = plsc.VectorSubcoreMesh(core_axis_name="c", subcore_axis_name="s")

    scratch = [pltpu.VMEM((n_chunks, CHUNK), jnp.int32)]
    scratch += [pltpu.VMEM((CHUNK, D), jnp.float32) for _ in range(NBUF)]
    scratch += [pltpu.VMEM((D // 8, 8, CHUNK + 1), jnp.float32)
                for _ in range(NBUF)]
    scratch += [pltpu.SemaphoreType.DMA, pltpu.SemaphoreType.DMA]

    @functools.partial(
        pl.kernel,
        mesh=mesh,
        compiler_params=pltpu.CompilerParams(
            use_tc_tiling_on_sc=False, needs_layout_passes=False
        ),
        out_type=jax.ShapeDtypeStruct((S, D // 8, RT, 8, CHUNK), jnp.float32),
        scratch_types=scratch,
    )
    def k(idx_hbm, table_hbm, out_hbm, idx_v, *bufs):
        rows_bufs = bufs[:NBUF]
        trans_bufs = bufs[NBUF:2 * NBUF]
        gsem, wsem = bufs[2 * NBUF], bufs[2 * NBUF + 1]

        wid = lax.axis_index("s") * NC + lax.axis_index("c")
        cbase = wid * n_chunks
        pltpu.sync_copy(idx_hbm.at[wid], idx_v)

        iota16 = lax.iota(jnp.int32, 16)
        td_vecs, di_vecs = [], []
        for dg in range(D // 16):
            dvec = iota16 + dg * 16
            td_vecs.append(lax.shift_right_logical(dvec, 3))
            di_vecs.append(lax.bitwise_and(dvec, 7))

        zero16 = iota16 * 0

        def transpose_chunk(buf):
            rows = rows_bufs[buf]
            trans = trans_bufs[buf]

            @plsc.parallel_loop(0, CHUNK, unroll=8)
            def _(r):
                rsplat = zero16 + r
                for dg in range(D // 16):
                    vec = rows[r, pl.ds(dg * 16, 16)]
                    plsc.store_scatter(
                        trans, [td_vecs[dg], di_vecs[dg], rsplat], vec
                    )

        def gather_group(q, bufset):
            for j in range(GRP):
                pltpu.async_copy(
                    table_hbm.at[idx_v.at[q * GRP + j]],
                    rows_bufs[bufset * GRP + j],
                    gsem,
                )

        def body(q, bufset):
            @pl.when(q >= 2)
            def _():
                for j in range(GRP):
                    pltpu.make_async_copy(
                        trans_bufs[j].at[:, :, pl.ds(0, CHUNK)], out_hbm.at[0, :, 0], wsem
                    ).wait()
            for j in range(GRP):
                pltpu.make_async_copy(
                    table_hbm.at[pl.ds(0, CHUNK)], rows_bufs[0], gsem
                ).wait()
            @pl.when(q + 1 < n_grp)
            def _():
                gather_group(q + 1, 1 - bufset)
            for j in range(GRP):
                buf = bufset * GRP + j
                transpose_chunk(buf)
                c = cbase + q * GRP + j
                s = c // RT
                tr = lax.rem(c, RT)
                pltpu.async_copy(
                    trans_bufs[buf].at[:, :, pl.ds(0, CHUNK)], out_hbm.at[s, :, tr], wsem
                )

        def outer(p, carry):
            for grp in range(2):
                body(p * 2 + grp, grp)
            return carry

        gather_group(0, 0)
        lax.fori_loop(0, n_grp // 2, outer, 0, unroll=False)
        for j in range(2 * GRP):
            pltpu.make_async_copy(
                trans_bufs[j % GRP].at[:, :, pl.ds(0, CHUNK)], out_hbm.at[0, :, 0], wsem
            ).wait()

    return k


def kernel(x, table):
    R, S = x.shape
    xt = x.T.astype(jnp.int32)
    idx3 = xt.reshape(NW, (S * R) // (NW * CHUNK), CHUNK)
    out5 = _build(S, R)(idx3, table)
    return out5.transpose(2, 4, 0, 1, 3).reshape(R, S, D)
